# speedup vs baseline: 1.0372x; 1.0372x over previous
import functools

import jax
import jax.numpy as jnp
from jax import lax
from jax.experimental import pallas as pl
from jax.experimental.pallas import tpu as pltpu
from jax.experimental.pallas import tpu_sc as plsc

BATCH = 4
SEQ = 2048
HIDDEN = 2048
N_ROWS = BATCH * SEQ
NUM_CORES = 2
NUM_SUBCORES = 16
NUM_WORKERS = NUM_CORES * NUM_SUBCORES
ROWS_PER_W = N_ROWS // NUM_WORKERS
CHUNK = 8
NUM_CHUNKS = ROWS_PER_W // CHUNK
LANES = 16
VECS_PER_ROW = HIDDEN // LANES

_mesh = plsc.VectorSubcoreMesh(core_axis_name="c", subcore_axis_name="s")


@functools.partial(
    pl.kernel,
    mesh=_mesh,
    out_type=jax.ShapeDtypeStruct((N_ROWS, HIDDEN), jnp.float32),
    scratch_types=[
        pltpu.VMEM((ROWS_PER_W,), jnp.int32),
        pltpu.VMEM((ROWS_PER_W,), jnp.int32),
        pltpu.VMEM((CHUNK, HIDDEN), jnp.float32),
        pltpu.VMEM((CHUNK, HIDDEN), jnp.float32),
        pltpu.SemaphoreType.DMA,
    ],
)
def _emb_kernel(idw_hbm, idp_hbm, wt_hbm, pt_hbm, out_hbm,
                idw_v, idp_v, wrow, prow, sem):
    wid = lax.axis_index("s") * NUM_CORES + lax.axis_index("c")
    base = pl.multiple_of(wid * ROWS_PER_W, ROWS_PER_W)
    pltpu.sync_copy(idw_hbm.at[pl.ds(base, ROWS_PER_W)], idw_v)
    pltpu.sync_copy(idp_hbm.at[pl.ds(base, ROWS_PER_W)], idp_v)

    def chunk_body(ci, _):
        off = pl.multiple_of(ci * CHUNK, CHUNK)
        cw = pltpu.async_copy(wt_hbm.at[idw_v.at[pl.ds(off, CHUNK)]], wrow, sem)
        cp = pltpu.async_copy(pt_hbm.at[idp_v.at[pl.ds(off, CHUNK)]], prow, sem)
        cw.wait()
        cp.wait()

        def vec_body(vi, _):
            col = pl.ds(vi * LANES, LANES)
            for r in range(CHUNK):
                wrow[r, col] = wrow[r, col] + prow[r, col]
            return 0

        lax.fori_loop(0, VECS_PER_ROW, vec_body, 0)
        pltpu.sync_copy(wrow, out_hbm.at[pl.ds(base + off, CHUNK)])
        return 0

    lax.fori_loop(0, NUM_CHUNKS, chunk_body, 0)


def kernel(input_ids, position_ids, word_table, pos_table):
    idw = jnp.transpose(input_ids).reshape(-1).astype(jnp.int32)
    idp = jnp.transpose(position_ids).reshape(-1).astype(jnp.int32)
    out = _emb_kernel(idw, idp, word_table, pos_table)
    return out.reshape(SEQ, BATCH, HIDDEN)


# speedup vs baseline: 1.5976x; 1.5403x over previous
import functools

import jax
import jax.numpy as jnp
from jax import lax
from jax.experimental import pallas as pl
from jax.experimental.pallas import tpu as pltpu
from jax.experimental.pallas import tpu_sc as plsc

BATCH = 4
SEQ = 2048
HIDDEN = 2048
N_ROWS = BATCH * SEQ
NUM_CORES = 2
NUM_SUBCORES = 16
NUM_WORKERS = NUM_CORES * NUM_SUBCORES
ROWS_PER_W = N_ROWS // NUM_WORKERS
CHUNK = 4
NBUF = 4
NUM_CHUNKS = ROWS_PER_W // CHUNK
NUM_STEPS = NUM_CHUNKS // NBUF
LANES = 16
VECS_PER_ROW = HIDDEN // LANES

_mesh = plsc.VectorSubcoreMesh(core_axis_name="c", subcore_axis_name="s")


@functools.partial(
    pl.kernel,
    mesh=_mesh,
    out_type=jax.ShapeDtypeStruct((N_ROWS, HIDDEN), jnp.float32),
    scratch_types=[
        pltpu.VMEM((NUM_CHUNKS, CHUNK), jnp.int32),
        pltpu.VMEM((NUM_CHUNKS, CHUNK), jnp.int32),
        [pltpu.VMEM((CHUNK, HIDDEN), jnp.float32) for _ in range(NBUF)],
        [pltpu.VMEM((CHUNK, HIDDEN), jnp.float32) for _ in range(NBUF)],
        [pltpu.SemaphoreType.DMA for _ in range(NBUF)],
        [pltpu.SemaphoreType.DMA for _ in range(NBUF)],
    ],
)
def _emb_kernel(idw_hbm, idp_hbm, wt_hbm, pt_hbm, out_hbm,
                idw_v, idp_v, wbufs, pbufs, gsems, ssems):
    wid = lax.axis_index("s") * NUM_CORES + lax.axis_index("c")
    base = pl.multiple_of(wid * ROWS_PER_W, ROWS_PER_W)
    pltpu.sync_copy(idw_hbm.at[wid], idw_v)
    pltpu.sync_copy(idp_hbm.at[wid], idp_v)

    def issue_gathers(ci, j):
        pltpu.async_copy(wt_hbm.at[idw_v.at[ci]], wbufs[j], gsems[j])
        pltpu.async_copy(pt_hbm.at[idp_v.at[ci]], pbufs[j], gsems[j])

    def wait_gathers(j):
        pltpu.make_async_copy(wt_hbm.at[idw_v.at[0]], wbufs[j], gsems[j]).wait()
        pltpu.make_async_copy(pt_hbm.at[idp_v.at[0]], pbufs[j], gsems[j]).wait()

    def wait_scatter(j):
        pltpu.make_async_copy(
            wbufs[j], out_hbm.at[pl.ds(base, CHUNK)], ssems[j]).wait()

    for j in range(NBUF):
        issue_gathers(j, j)

    def step(pi, _):
        for j in range(NBUF):
            ci = pi * NBUF + j
            wait_gathers(j)
            wbuf, pbuf = wbufs[j], pbufs[j]

            def vec_body(vi, _):
                col = pl.ds(vi * LANES, LANES)
                for r in range(CHUNK):
                    wbuf[r, col] = wbuf[r, col] + pbuf[r, col]
                return 0

            lax.fori_loop(0, VECS_PER_ROW, vec_body, 0)
            pltpu.async_copy(
                wbuf, out_hbm.at[pl.ds(base + ci * CHUNK, CHUNK)], ssems[j])

            nxt = ci + NBUF

            @pl.when(nxt < NUM_CHUNKS)
            def _():
                wait_scatter(j)
                issue_gathers(nxt, j)

        return 0

    lax.fori_loop(0, NUM_STEPS, step, 0)
    for j in range(NBUF):
        wait_scatter(j)


def kernel(input_ids, position_ids, word_table, pos_table):
    idw = jnp.transpose(input_ids).reshape(NUM_WORKERS, NUM_CHUNKS, CHUNK)
    idp = jnp.transpose(position_ids).reshape(NUM_WORKERS, NUM_CHUNKS, CHUNK)
    out = _emb_kernel(idw.astype(jnp.int32), idp.astype(jnp.int32),
                      word_table, pos_table)
    return out.reshape(SEQ, BATCH, HIDDEN)
